# baseline (device time: 46223 ns/iter reference)
import jax
import jax.numpy as jnp
from jax import lax
from jax.experimental import pallas as pl
from jax.experimental.pallas import tpu as pltpu

N_Z = 4


def kernel(dy, W):
    m, k = dy.shape
    n = W.shape[0]

    def body(dy_ref, w_ref, out_ref, acc_ref, comm_ref, send_sems, recv_sems):
        my_x = lax.axis_index("x")
        my_y = lax.axis_index("y")
        my_z = lax.axis_index("z")

        acc_ref[...] = lax.dot_general(
            dy_ref[...],
            w_ref[...],
            dimension_numbers=(((1,), (1,)), ((), ())),
            preferred_element_type=jnp.float32,
        )

        barrier_sem = pltpu.get_barrier_semaphore()
        for d in range(1, N_Z):
            peer_z = (my_z + d) % N_Z
            pl.semaphore_signal(
                barrier_sem,
                inc=1,
                device_id=(my_x, my_y, peer_z),
                device_id_type=pl.DeviceIdType.MESH,
            )
        pl.semaphore_wait(barrier_sem, N_Z - 1)

        rdmas = []
        for d in range(1, N_Z):
            peer_z = (my_z + d) % N_Z
            rdma = pltpu.make_async_remote_copy(
                src_ref=acc_ref,
                dst_ref=comm_ref.at[3 - d],
                send_sem=send_sems.at[d - 1],
                recv_sem=recv_sems.at[3 - d],
                device_id=(my_x, my_y, peer_z),
                device_id_type=pl.DeviceIdType.MESH,
            )
            rdma.start()
            rdmas.append(rdma)
        for rdma in rdmas:
            rdma.wait()

        out_ref[...] = acc_ref[...] + comm_ref[0] + comm_ref[1] + comm_ref[2]

    return pl.pallas_call(
        body,
        out_shape=jax.ShapeDtypeStruct((m, n), jnp.float32),
        in_specs=[
            pl.BlockSpec(memory_space=pltpu.VMEM),
            pl.BlockSpec(memory_space=pltpu.VMEM),
        ],
        out_specs=pl.BlockSpec(memory_space=pltpu.VMEM),
        scratch_shapes=[
            pltpu.VMEM((m, n), jnp.float32),
            pltpu.VMEM((3, m, n), jnp.float32),
            pltpu.SemaphoreType.DMA((3,)),
            pltpu.SemaphoreType.DMA((3,)),
        ],
        compiler_params=pltpu.CompilerParams(collective_id=0),
    )(dy, W)


# device time: 30554 ns/iter; 1.5128x vs baseline; 1.5128x over previous
import jax
import jax.numpy as jnp
from jax import lax
from jax.experimental import pallas as pl
from jax.experimental.pallas import tpu as pltpu

N_Z = 4
QM = 128


def kernel(dy, W):
    m, k = dy.shape
    n = W.shape[0]

    def body(dy_ref, w_ref, out_ref, acc_ref, psum_ref, oq_ref,
             zbuf_ref, xybuf_ref, send_sems, zrecv_sems, xyrecv_sems):
        my_x = lax.axis_index("x")
        my_y = lax.axis_index("y")
        my_z = lax.axis_index("z")
        c = 2 * my_x + my_y

        is_z0 = my_z == 0
        is_z1 = my_z == 1
        is_z3 = my_z == N_Z - 1
        is_edge = jnp.logical_or(is_z0, is_z3)
        is_mid = jnp.logical_not(is_edge)

        pair_z = my_z + jnp.where(jnp.logical_or(is_z0, my_z == 2), 1, -1)

        acc_ref[...] = lax.dot_general(
            dy_ref[pl.ds(c * QM, QM), :],
            w_ref[...],
            dimension_numbers=(((1,), (1,)), ((), ())),
            preferred_element_type=jnp.float32,
        )

        barrier_sem = pltpu.get_barrier_semaphore()
        for dev in (
            (1 - my_x, my_y, my_z),
            (my_x, 1 - my_y, my_z),
            (1 - my_x, 1 - my_y, my_z),
            (my_x, my_y, pair_z),
        ):
            pl.semaphore_signal(
                barrier_sem, inc=1, device_id=dev,
                device_id_type=pl.DeviceIdType.MESH,
            )

        far_z = jnp.where(is_z0, 2, jnp.where(is_z3, 1, jnp.where(is_z1, 3, 0)))
        pl.semaphore_signal(
            barrier_sem, inc=1, device_id=(my_x, my_y, far_z),
            device_id_type=pl.DeviceIdType.MESH,
        )
        @pl.when(is_mid)
        def _():
            other_mid = jnp.where(is_z1, 2, 1)
            pl.semaphore_signal(
                barrier_sem, inc=1, device_id=(my_x, my_y, other_mid),
                device_id_type=pl.DeviceIdType.MESH,
            )

        pl.semaphore_wait(barrier_sem, 5)
        @pl.when(is_mid)
        def _():
            pl.semaphore_wait(barrier_sem, 1)

        zr1 = pltpu.make_async_remote_copy(
            src_ref=acc_ref,
            dst_ref=zbuf_ref.at[0],
            send_sem=send_sems.at[0],
            recv_sem=zrecv_sems.at[0],
            device_id=(my_x, my_y, pair_z),
            device_id_type=pl.DeviceIdType.MESH,
        )
        zr1.start()
        zr1.wait()
        psum_ref[...] = acc_ref[...] + zbuf_ref[0]

        @pl.when(is_mid)
        def _():
            other_mid = jnp.where(is_z1, 2, 1)
            far_edge = jnp.where(is_z1, 3, 0)
            s_near = pltpu.make_async_remote_copy(
                src_ref=psum_ref,
                dst_ref=zbuf_ref.at[1],
                send_sem=send_sems.at[1],
                recv_sem=zrecv_sems.at[1],
                device_id=(my_x, my_y, other_mid),
                device_id_type=pl.DeviceIdType.MESH,
            )
            s_far = pltpu.make_async_remote_copy(
                src_ref=psum_ref,
                dst_ref=zbuf_ref.at[1],
                send_sem=send_sems.at[2],
                recv_sem=zrecv_sems.at[1],
                device_id=(my_x, my_y, far_edge),
                device_id_type=pl.DeviceIdType.MESH,
            )
            s_near.start()
            s_far.start()
            s_near.wait()
            s_far.wait_send()

        @pl.when(is_edge)
        def _():
            recv = pltpu.make_async_remote_copy(
                src_ref=psum_ref,
                dst_ref=zbuf_ref.at[1],
                send_sem=send_sems.at[1],
                recv_sem=zrecv_sems.at[1],
                device_id=(my_x, my_y, pair_z),
                device_id_type=pl.DeviceIdType.MESH,
            )
            recv.wait_recv()

        oq_ref[...] = psum_ref[...] + zbuf_ref[1]

        xs = pltpu.make_async_remote_copy(
            src_ref=oq_ref,
            dst_ref=xybuf_ref.at[0],
            send_sem=send_sems.at[3],
            recv_sem=xyrecv_sems.at[0],
            device_id=(1 - my_x, my_y, my_z),
            device_id_type=pl.DeviceIdType.MESH,
        )
        ys = pltpu.make_async_remote_copy(
            src_ref=oq_ref,
            dst_ref=xybuf_ref.at[1],
            send_sem=send_sems.at[4],
            recv_sem=xyrecv_sems.at[1],
            device_id=(my_x, 1 - my_y, my_z),
            device_id_type=pl.DeviceIdType.MESH,
        )
        ds = pltpu.make_async_remote_copy(
            src_ref=oq_ref,
            dst_ref=xybuf_ref.at[2],
            send_sem=send_sems.at[5],
            recv_sem=xyrecv_sems.at[2],
            device_id=(1 - my_x, 1 - my_y, my_z),
            device_id_type=pl.DeviceIdType.MESH,
        )
        xs.start()
        ys.start()
        ds.start()

        out_ref[pl.ds(c * QM, QM), :] = oq_ref[...]

        xs.wait()
        out_ref[pl.ds((2 * (1 - my_x) + my_y) * QM, QM), :] = xybuf_ref[0]
        ys.wait()
        out_ref[pl.ds((2 * my_x + (1 - my_y)) * QM, QM), :] = xybuf_ref[1]
        ds.wait()
        out_ref[pl.ds((2 * (1 - my_x) + (1 - my_y)) * QM, QM), :] = xybuf_ref[2]

    return pl.pallas_call(
        body,
        out_shape=jax.ShapeDtypeStruct((m, n), jnp.float32),
        in_specs=[
            pl.BlockSpec(memory_space=pltpu.VMEM),
            pl.BlockSpec(memory_space=pltpu.VMEM),
        ],
        out_specs=pl.BlockSpec(memory_space=pltpu.VMEM),
        scratch_shapes=[
            pltpu.VMEM((QM, n), jnp.float32),
            pltpu.VMEM((QM, n), jnp.float32),
            pltpu.VMEM((QM, n), jnp.float32),
            pltpu.VMEM((2, QM, n), jnp.float32),
            pltpu.VMEM((3, QM, n), jnp.float32),
            pltpu.SemaphoreType.DMA((6,)),
            pltpu.SemaphoreType.DMA((2,)),
            pltpu.SemaphoreType.DMA((3,)),
        ],
        compiler_params=pltpu.CompilerParams(collective_id=0),
    )(dy, W)


# device time: 26710 ns/iter; 1.7306x vs baseline; 1.1439x over previous
import jax
import jax.numpy as jnp
from jax import lax
from jax.experimental import pallas as pl
from jax.experimental.pallas import tpu as pltpu

N_Z = 4
QM = 128
HM = 64


def kernel(dy, W):
    m, k = dy.shape
    n = W.shape[0]

    def body(dy_hbm, w_hbm, out_ref, dyq_ref, w_ref, acc_ref, psum_ref,
             oq_ref, zbuf_ref, xybuf_ref, in_sems, send_sems, zrecv_sems,
             xyrecv_sems):
        my_x = lax.axis_index("x")
        my_y = lax.axis_index("y")
        my_z = lax.axis_index("z")
        c = 2 * my_x + my_y

        is_z0 = my_z == 0
        is_z1 = my_z == 1
        is_z3 = my_z == N_Z - 1
        is_edge = jnp.logical_or(is_z0, is_z3)
        is_mid = jnp.logical_not(is_edge)

        pair_z = my_z + jnp.where(jnp.logical_or(is_z0, my_z == 2), 1, -1)
        other_mid = jnp.where(is_z1, 2, 1)
        far_edge = jnp.where(is_z1, 3, 0)

        def rcopy(src, dst, ssem, rsem, dev):
            return pltpu.make_async_remote_copy(
                src_ref=src, dst_ref=dst, send_sem=ssem, recv_sem=rsem,
                device_id=dev, device_id_type=pl.DeviceIdType.MESH,
            )

        dcp = pltpu.make_async_copy(
            dy_hbm.at[pl.ds(c * QM, QM), :], dyq_ref, in_sems.at[0])
        wcp = pltpu.make_async_copy(w_hbm, w_ref, in_sems.at[1])
        dcp.start()
        wcp.start()

        barrier_sem = pltpu.get_barrier_semaphore()
        far_z = jnp.where(is_z0, 2, jnp.where(is_z3, 1, jnp.where(is_z1, 3, 0)))
        for dev in (
            (1 - my_x, my_y, my_z),
            (my_x, 1 - my_y, my_z),
            (1 - my_x, 1 - my_y, my_z),
            (my_x, my_y, pair_z),
            (my_x, my_y, far_z),
        ):
            pl.semaphore_signal(
                barrier_sem, inc=1, device_id=dev,
                device_id_type=pl.DeviceIdType.MESH,
            )

        @pl.when(is_mid)
        def _():
            pl.semaphore_signal(
                barrier_sem, inc=1, device_id=(my_x, my_y, other_mid),
                device_id_type=pl.DeviceIdType.MESH,
            )

        pl.semaphore_wait(barrier_sem, 5)
        @pl.when(is_mid)
        def _():
            pl.semaphore_wait(barrier_sem, 1)

        dcp.wait()
        wcp.wait()

        for h in range(2):
            acc_ref[h] = lax.dot_general(
                dyq_ref[pl.ds(h * HM, HM), :],
                w_ref[...],
                dimension_numbers=(((1,), (1,)), ((), ())),
                preferred_element_type=jnp.float32,
            )
            rcopy(acc_ref.at[h], zbuf_ref.at[h, 0], send_sems.at[h, 0],
                  zrecv_sems.at[h, 0], (my_x, my_y, pair_z)).start()

        for h in range(2):
            rcopy(acc_ref.at[h], zbuf_ref.at[h, 0], send_sems.at[h, 0],
                  zrecv_sems.at[h, 0], (my_x, my_y, pair_z)).wait()
            psum_ref[h] = acc_ref[h] + zbuf_ref[h, 0]

            @pl.when(is_mid)
            def _(h=h):
                rcopy(psum_ref.at[h], zbuf_ref.at[h, 1], send_sems.at[h, 1],
                      zrecv_sems.at[h, 1], (my_x, my_y, other_mid)).start()
                rcopy(psum_ref.at[h], zbuf_ref.at[h, 1], send_sems.at[h, 2],
                      zrecv_sems.at[h, 1], (my_x, my_y, far_edge)).start()

        xy_devs = (
            (1 - my_x, my_y, my_z),
            (my_x, 1 - my_y, my_z),
            (1 - my_x, 1 - my_y, my_z),
        )
        for h in range(2):
            rcopy(psum_ref.at[h], zbuf_ref.at[h, 1], send_sems.at[h, 1],
                  zrecv_sems.at[h, 1], (my_x, my_y, pair_z)).wait_recv()
            oq_ref[h] = psum_ref[h] + zbuf_ref[h, 1]
            for j, dev in enumerate(xy_devs):
                rcopy(oq_ref.at[h], xybuf_ref.at[h, j], send_sems.at[h, 3 + j],
                      xyrecv_sems.at[h, j], dev).start()
            out_ref[pl.ds(c * QM + h * HM, HM), :] = oq_ref[h]

        src_cols = (
            2 * (1 - my_x) + my_y,
            2 * my_x + (1 - my_y),
            2 * (1 - my_x) + (1 - my_y),
        )
        for h in range(2):
            for j, (dev, cc) in enumerate(zip(xy_devs, src_cols)):
                rcopy(oq_ref.at[h], xybuf_ref.at[h, j], send_sems.at[h, 3 + j],
                      xyrecv_sems.at[h, j], dev).wait_recv()
                out_ref[pl.ds(cc * QM + h * HM, HM), :] = xybuf_ref[h, j]

        for h in range(2):
            for j, dev in enumerate(xy_devs):
                rcopy(oq_ref.at[h], xybuf_ref.at[h, j], send_sems.at[h, 3 + j],
                      xyrecv_sems.at[h, j], dev).wait_send()

            @pl.when(is_mid)
            def _(h=h):
                rcopy(psum_ref.at[h], zbuf_ref.at[h, 1], send_sems.at[h, 1],
                      zrecv_sems.at[h, 1], (my_x, my_y, other_mid)).wait_send()
                rcopy(psum_ref.at[h], zbuf_ref.at[h, 1], send_sems.at[h, 2],
                      zrecv_sems.at[h, 1], (my_x, my_y, far_edge)).wait_send()

    return pl.pallas_call(
        body,
        out_shape=jax.ShapeDtypeStruct((m, n), jnp.float32),
        in_specs=[
            pl.BlockSpec(memory_space=pl.ANY),
            pl.BlockSpec(memory_space=pl.ANY),
        ],
        out_specs=pl.BlockSpec(memory_space=pltpu.VMEM),
        scratch_shapes=[
            pltpu.VMEM((QM, k), jnp.float32),
            pltpu.VMEM((n, k), jnp.float32),
            pltpu.VMEM((2, HM, n), jnp.float32),
            pltpu.VMEM((2, HM, n), jnp.float32),
            pltpu.VMEM((2, HM, n), jnp.float32),
            pltpu.VMEM((2, 2, HM, n), jnp.float32),
            pltpu.VMEM((2, 3, HM, n), jnp.float32),
            pltpu.SemaphoreType.DMA((2,)),
            pltpu.SemaphoreType.DMA((2, 6)),
            pltpu.SemaphoreType.DMA((2, 2)),
            pltpu.SemaphoreType.DMA((2, 3)),
        ],
        compiler_params=pltpu.CompilerParams(collective_id=0),
    )(dy, W)


# device time: 26583 ns/iter; 1.7388x vs baseline; 1.0048x over previous
import jax
import jax.numpy as jnp
from jax import lax
from jax.experimental import pallas as pl
from jax.experimental.pallas import tpu as pltpu

N_Z = 4
QM = 128
HM = 64


def kernel(dy, W):
    m, k = dy.shape
    n = W.shape[0]

    def body(dy_hbm, w_hbm, out_hbm, dyq_ref, w_ref, acc_ref, psum_ref,
             oq_ref, zbuf_ref, xybuf_ref, in_sems, send_sems, zrecv_sems,
             xyrecv_sems, ostore_sems, zr2_sem, xy_sem):
        my_x = lax.axis_index("x")
        my_y = lax.axis_index("y")
        my_z = lax.axis_index("z")
        c = 2 * my_x + my_y

        is_z0 = my_z == 0
        is_z1 = my_z == 1
        is_z3 = my_z == N_Z - 1
        is_edge = jnp.logical_or(is_z0, is_z3)
        is_mid = jnp.logical_not(is_edge)

        pair_z = my_z + jnp.where(jnp.logical_or(is_z0, my_z == 2), 1, -1)
        other_mid = jnp.where(is_z1, 2, 1)
        far_edge = jnp.where(is_z1, 3, 0)
        writer_mid = jnp.where(my_z <= 1, 2, 1)

        def rcopy(src, dst, ssem, rsem, dev):
            return pltpu.make_async_remote_copy(
                src_ref=src, dst_ref=dst, send_sem=ssem, recv_sem=rsem,
                device_id=dev, device_id_type=pl.DeviceIdType.MESH,
            )

        dcp = pltpu.make_async_copy(
            dy_hbm.at[pl.ds(c * QM, QM), :], dyq_ref, in_sems.at[0])
        wcp = pltpu.make_async_copy(w_hbm, w_ref, in_sems.at[1])
        dcp.start()
        wcp.start()

        barrier_sem = pltpu.get_barrier_semaphore()
        pl.semaphore_signal(
            barrier_sem, inc=1, device_id=(my_x, my_y, pair_z),
            device_id_type=pl.DeviceIdType.MESH,
        )
        pl.semaphore_signal(
            zr2_sem, inc=1, device_id=(my_x, my_y, writer_mid),
            device_id_type=pl.DeviceIdType.MESH,
        )
        xy_devs = (
            (1 - my_x, my_y, my_z),
            (my_x, 1 - my_y, my_z),
            (1 - my_x, 1 - my_y, my_z),
        )
        for dev in xy_devs:
            pl.semaphore_signal(
                xy_sem, inc=1, device_id=dev,
                device_id_type=pl.DeviceIdType.MESH,
            )

        dcp.wait()
        wcp.wait()

        for h in range(2):
            acc_ref[h] = lax.dot_general(
                dyq_ref[pl.ds(h * HM, HM), :],
                w_ref[...],
                dimension_numbers=(((1,), (1,)), ((), ())),
                preferred_element_type=jnp.float32,
            )
            if h == 0:
                pl.semaphore_wait(barrier_sem, 1)
            rcopy(acc_ref.at[h], zbuf_ref.at[h, 0], send_sems.at[h, 0],
                  zrecv_sems.at[h, 0], (my_x, my_y, pair_z)).start()

        for h in range(2):
            rcopy(acc_ref.at[h], zbuf_ref.at[h, 0], send_sems.at[h, 0],
                  zrecv_sems.at[h, 0], (my_x, my_y, pair_z)).wait()
            psum_ref[h] = acc_ref[h] + zbuf_ref[h, 0]

            @pl.when(is_mid)
            def _(h=h):
                if h == 0:
                    pl.semaphore_wait(zr2_sem, 2)
                rcopy(psum_ref.at[h], zbuf_ref.at[h, 1], send_sems.at[h, 1],
                      zrecv_sems.at[h, 1], (my_x, my_y, other_mid)).start()
                rcopy(psum_ref.at[h], zbuf_ref.at[h, 1], send_sems.at[h, 2],
                      zrecv_sems.at[h, 1], (my_x, my_y, far_edge)).start()

        ostores = []
        for h in range(2):
            rcopy(psum_ref.at[h], zbuf_ref.at[h, 1], send_sems.at[h, 1],
                  zrecv_sems.at[h, 1], (my_x, my_y, pair_z)).wait_recv()
            oq_ref[h] = psum_ref[h] + zbuf_ref[h, 1]
            if h == 0:
                pl.semaphore_wait(xy_sem, 3)
            for j, dev in enumerate(xy_devs):
                rcopy(oq_ref.at[h], xybuf_ref.at[h, j], send_sems.at[h, 3 + j],
                      xyrecv_sems.at[h, j], dev).start()
            st = pltpu.make_async_copy(
                oq_ref.at[h],
                out_hbm.at[pl.ds(c * QM + h * HM, HM), :],
                ostore_sems.at[h, 0])
            st.start()
            ostores.append(st)

        src_cols = (
            2 * (1 - my_x) + my_y,
            2 * my_x + (1 - my_y),
            2 * (1 - my_x) + (1 - my_y),
        )
        for h in range(2):
            for j, (dev, cc) in enumerate(zip(xy_devs, src_cols)):
                rcopy(oq_ref.at[h], xybuf_ref.at[h, j], send_sems.at[h, 3 + j],
                      xyrecv_sems.at[h, j], dev).wait_recv()
                st = pltpu.make_async_copy(
                    xybuf_ref.at[h, j],
                    out_hbm.at[pl.ds(cc * QM + h * HM, HM), :],
                    ostore_sems.at[h, 1 + j])
                st.start()
                ostores.append(st)

        for st in ostores:
            st.wait()
        for h in range(2):
            for j, dev in enumerate(xy_devs):
                rcopy(oq_ref.at[h], xybuf_ref.at[h, j], send_sems.at[h, 3 + j],
                      xyrecv_sems.at[h, j], dev).wait_send()

            @pl.when(is_mid)
            def _(h=h):
                rcopy(psum_ref.at[h], zbuf_ref.at[h, 1], send_sems.at[h, 1],
                      zrecv_sems.at[h, 1], (my_x, my_y, other_mid)).wait_send()
                rcopy(psum_ref.at[h], zbuf_ref.at[h, 1], send_sems.at[h, 2],
                      zrecv_sems.at[h, 1], (my_x, my_y, far_edge)).wait_send()

    return pl.pallas_call(
        body,
        out_shape=jax.ShapeDtypeStruct((m, n), jnp.float32),
        in_specs=[
            pl.BlockSpec(memory_space=pl.ANY),
            pl.BlockSpec(memory_space=pl.ANY),
        ],
        out_specs=pl.BlockSpec(memory_space=pl.ANY),
        scratch_shapes=[
            pltpu.VMEM((QM, k), jnp.float32),
            pltpu.VMEM((n, k), jnp.float32),
            pltpu.VMEM((2, HM, n), jnp.float32),
            pltpu.VMEM((2, HM, n), jnp.float32),
            pltpu.VMEM((2, HM, n), jnp.float32),
            pltpu.VMEM((2, 2, HM, n), jnp.float32),
            pltpu.VMEM((2, 3, HM, n), jnp.float32),
            pltpu.SemaphoreType.DMA((2,)),
            pltpu.SemaphoreType.DMA((2, 6)),
            pltpu.SemaphoreType.DMA((2, 2)),
            pltpu.SemaphoreType.DMA((2, 3)),
            pltpu.SemaphoreType.DMA((2, 4)),
            pltpu.SemaphoreType.REGULAR,
            pltpu.SemaphoreType.REGULAR,
        ],
        compiler_params=pltpu.CompilerParams(collective_id=0),
    )(dy, W)


# device time: 24382 ns/iter; 1.8958x vs baseline; 1.0903x over previous
import jax
import jax.numpy as jnp
from jax import lax
from jax.experimental import pallas as pl
from jax.experimental.pallas import tpu as pltpu

N_Z = 4
QM = 128
WAVES = 2
HM = QM // WAVES


def kernel(dy, W):
    m, k = dy.shape
    n = W.shape[0]

    def body(dy_hbm, w_hbm, out_hbm, dyq_ref, w_ref, acc_ref, psum_ref,
             oq_ref, zbuf_ref, xybuf_ref, in_sems, send_sems, zrecv_sems,
             xyrecv_sems, ostore_sems, zr2_sem, xy_sem):
        my_x = lax.axis_index("x")
        my_y = lax.axis_index("y")
        my_z = lax.axis_index("z")
        c = 2 * my_x + my_y

        is_z0 = my_z == 0
        is_z1 = my_z == 1
        is_z3 = my_z == N_Z - 1
        is_edge = jnp.logical_or(is_z0, is_z3)
        is_mid = jnp.logical_not(is_edge)

        pair_z = my_z + jnp.where(jnp.logical_or(is_z0, my_z == 2), 1, -1)
        other_mid = jnp.where(is_z1, 2, 1)
        far_edge = jnp.where(is_z1, 3, 0)
        writer_mid = jnp.where(my_z <= 1, 2, 1)

        def rcopy(src, dst, ssem, rsem, dev):
            return pltpu.make_async_remote_copy(
                src_ref=src, dst_ref=dst, send_sem=ssem, recv_sem=rsem,
                device_id=dev, device_id_type=pl.DeviceIdType.MESH,
            )

        dcp = pltpu.make_async_copy(
            dy_hbm.at[pl.ds(c * QM, QM), :], dyq_ref, in_sems.at[0])
        wcp = pltpu.make_async_copy(w_hbm, w_ref, in_sems.at[1])
        dcp.start()
        wcp.start()

        barrier_sem = pltpu.get_barrier_semaphore()
        pl.semaphore_signal(
            barrier_sem, inc=1, device_id=(my_x, my_y, pair_z),
            device_id_type=pl.DeviceIdType.MESH,
        )
        pl.semaphore_signal(
            zr2_sem, inc=1, device_id=(my_x, my_y, writer_mid),
            device_id_type=pl.DeviceIdType.MESH,
        )
        xy_devs = (
            (1 - my_x, 1 - my_y, my_z),
            (1 - my_x, my_y, my_z),
            (my_x, 1 - my_y, my_z),
        )
        for dev in xy_devs:
            pl.semaphore_signal(
                xy_sem, inc=1, device_id=dev,
                device_id_type=pl.DeviceIdType.MESH,
            )

        dcp.wait()
        wcp.wait()

        for h in range(WAVES):
            acc_ref[h] = lax.dot_general(
                dyq_ref[pl.ds(h * HM, HM), :],
                w_ref[...],
                dimension_numbers=(((1,), (1,)), ((), ())),
                preferred_element_type=jnp.float32,
            )
            if h == 0:
                pl.semaphore_wait(barrier_sem, 1)
            rcopy(acc_ref.at[h], zbuf_ref.at[h, 0], send_sems.at[h, 0],
                  zrecv_sems.at[h, 0], (my_x, my_y, pair_z)).start()

        for h in range(WAVES):
            rcopy(acc_ref.at[h], zbuf_ref.at[h, 0], send_sems.at[h, 0],
                  zrecv_sems.at[h, 0], (my_x, my_y, pair_z)).wait()
            psum_ref[h] = acc_ref[h] + zbuf_ref[h, 0]

            @pl.when(is_mid)
            def _(h=h):
                if h == 0:
                    pl.semaphore_wait(zr2_sem, 2)
                rcopy(psum_ref.at[h], zbuf_ref.at[h, 1], send_sems.at[h, 2],
                      zrecv_sems.at[h, 1], (my_x, my_y, far_edge)).start()
                rcopy(psum_ref.at[h], zbuf_ref.at[h, 1], send_sems.at[h, 1],
                      zrecv_sems.at[h, 1], (my_x, my_y, other_mid)).start()

        ostores = []
        for h in range(WAVES):
            rcopy(psum_ref.at[h], zbuf_ref.at[h, 1], send_sems.at[h, 1],
                  zrecv_sems.at[h, 1], (my_x, my_y, pair_z)).wait_recv()
            oq_ref[h] = psum_ref[h] + zbuf_ref[h, 1]
            if h == 0:
                pl.semaphore_wait(xy_sem, 3)
            for j, dev in enumerate(xy_devs):
                rcopy(oq_ref.at[h], xybuf_ref.at[h, j], send_sems.at[h, 3 + j],
                      xyrecv_sems.at[h, j], dev).start()
            st = pltpu.make_async_copy(
                oq_ref.at[h],
                out_hbm.at[pl.ds(c * QM + h * HM, HM), :],
                ostore_sems.at[h, 0])
            st.start()
            ostores.append(st)

        src_cols = (
            2 * (1 - my_x) + (1 - my_y),
            2 * (1 - my_x) + my_y,
            2 * my_x + (1 - my_y),
        )
        for h in range(WAVES):
            for j, (dev, cc) in enumerate(zip(xy_devs, src_cols)):
                rcopy(oq_ref.at[h], xybuf_ref.at[h, j], send_sems.at[h, 3 + j],
                      xyrecv_sems.at[h, j], dev).wait_recv()
                st = pltpu.make_async_copy(
                    xybuf_ref.at[h, j],
                    out_hbm.at[pl.ds(cc * QM + h * HM, HM), :],
                    ostore_sems.at[h, 1 + j])
                st.start()
                ostores.append(st)

        for st in ostores:
            st.wait()
        for h in range(WAVES):
            for j, dev in enumerate(xy_devs):
                rcopy(oq_ref.at[h], xybuf_ref.at[h, j], send_sems.at[h, 3 + j],
                      xyrecv_sems.at[h, j], dev).wait_send()

            @pl.when(is_mid)
            def _(h=h):
                rcopy(psum_ref.at[h], zbuf_ref.at[h, 1], send_sems.at[h, 1],
                      zrecv_sems.at[h, 1], (my_x, my_y, other_mid)).wait_send()
                rcopy(psum_ref.at[h], zbuf_ref.at[h, 1], send_sems.at[h, 2],
                      zrecv_sems.at[h, 1], (my_x, my_y, far_edge)).wait_send()

    return pl.pallas_call(
        body,
        out_shape=jax.ShapeDtypeStruct((m, n), jnp.float32),
        in_specs=[
            pl.BlockSpec(memory_space=pl.ANY),
            pl.BlockSpec(memory_space=pl.ANY),
        ],
        out_specs=pl.BlockSpec(memory_space=pl.ANY),
        scratch_shapes=[
            pltpu.VMEM((QM, k), jnp.float32),
            pltpu.VMEM((n, k), jnp.float32),
            pltpu.VMEM((WAVES, HM, n), jnp.float32),
            pltpu.VMEM((WAVES, HM, n), jnp.float32),
            pltpu.VMEM((WAVES, HM, n), jnp.float32),
            pltpu.VMEM((WAVES, 2, HM, n), jnp.float32),
            pltpu.VMEM((WAVES, 3, HM, n), jnp.float32),
            pltpu.SemaphoreType.DMA((2,)),
            pltpu.SemaphoreType.DMA((WAVES, 6)),
            pltpu.SemaphoreType.DMA((WAVES, 2)),
            pltpu.SemaphoreType.DMA((WAVES, 3)),
            pltpu.SemaphoreType.DMA((WAVES, 4)),
            pltpu.SemaphoreType.REGULAR,
            pltpu.SemaphoreType.REGULAR,
        ],
        compiler_params=pltpu.CompilerParams(collective_id=0),
    )(dy, W)


# device time: 23422 ns/iter; 1.9735x vs baseline; 1.0410x over previous
import jax
import jax.numpy as jnp
from jax import lax
from jax.experimental import pallas as pl
from jax.experimental.pallas import tpu as pltpu

N_Z = 4
QM = 128
WAVES = 4
HM = QM // WAVES


def kernel(dy, W):
    m, k = dy.shape
    n = W.shape[0]

    def body(dy_hbm, w_hbm, out_hbm, dyq_ref, w_ref, acc_ref, psum_ref,
             oq_ref, zbuf_ref, xybuf_ref, in_sems, send_sems, zrecv_sems,
             xyrecv_sems, ostore_sems, zr2_sem, xy_sem):
        my_x = lax.axis_index("x")
        my_y = lax.axis_index("y")
        my_z = lax.axis_index("z")
        c = 2 * my_x + my_y

        is_z0 = my_z == 0
        is_z1 = my_z == 1
        is_z3 = my_z == N_Z - 1
        is_edge = jnp.logical_or(is_z0, is_z3)
        is_mid = jnp.logical_not(is_edge)

        pair_z = my_z + jnp.where(jnp.logical_or(is_z0, my_z == 2), 1, -1)
        other_mid = jnp.where(is_z1, 2, 1)
        far_edge = jnp.where(is_z1, 3, 0)
        writer_mid = jnp.where(my_z <= 1, 2, 1)

        def rcopy(src, dst, ssem, rsem, dev):
            return pltpu.make_async_remote_copy(
                src_ref=src, dst_ref=dst, send_sem=ssem, recv_sem=rsem,
                device_id=dev, device_id_type=pl.DeviceIdType.MESH,
            )

        dcp = pltpu.make_async_copy(
            dy_hbm.at[pl.ds(c * QM, QM), :], dyq_ref, in_sems.at[0])
        wcp = pltpu.make_async_copy(w_hbm, w_ref, in_sems.at[1])
        dcp.start()
        wcp.start()

        barrier_sem = pltpu.get_barrier_semaphore()
        pl.semaphore_signal(
            barrier_sem, inc=1, device_id=(my_x, my_y, pair_z),
            device_id_type=pl.DeviceIdType.MESH,
        )
        pl.semaphore_signal(
            zr2_sem, inc=1, device_id=(my_x, my_y, writer_mid),
            device_id_type=pl.DeviceIdType.MESH,
        )
        xy_devs = (
            (1 - my_x, 1 - my_y, my_z),
            (1 - my_x, my_y, my_z),
            (my_x, 1 - my_y, my_z),
        )
        for dev in xy_devs:
            pl.semaphore_signal(
                xy_sem, inc=1, device_id=dev,
                device_id_type=pl.DeviceIdType.MESH,
            )

        dcp.wait()
        wcp.wait()

        for h in range(WAVES):
            acc_ref[h] = lax.dot_general(
                dyq_ref[pl.ds(h * HM, HM), :],
                w_ref[...],
                dimension_numbers=(((1,), (1,)), ((), ())),
                preferred_element_type=jnp.float32,
            )
            if h == 0:
                pl.semaphore_wait(barrier_sem, 1)
            rcopy(acc_ref.at[h], zbuf_ref.at[h, 0], send_sems.at[h, 0],
                  zrecv_sems.at[h, 0], (my_x, my_y, pair_z)).start()

        for h in range(WAVES):
            rcopy(acc_ref.at[h], zbuf_ref.at[h, 0], send_sems.at[h, 0],
                  zrecv_sems.at[h, 0], (my_x, my_y, pair_z)).wait()
            psum_ref[h] = acc_ref[h] + zbuf_ref[h, 0]

            @pl.when(is_mid)
            def _(h=h):
                if h == 0:
                    pl.semaphore_wait(zr2_sem, 2)
                rcopy(psum_ref.at[h], zbuf_ref.at[h, 1], send_sems.at[h, 2],
                      zrecv_sems.at[h, 1], (my_x, my_y, far_edge)).start()
                rcopy(psum_ref.at[h], zbuf_ref.at[h, 1], send_sems.at[h, 1],
                      zrecv_sems.at[h, 1], (my_x, my_y, other_mid)).start()

        ostores = []
        for h in range(WAVES):
            rcopy(psum_ref.at[h], zbuf_ref.at[h, 1], send_sems.at[h, 1],
                  zrecv_sems.at[h, 1], (my_x, my_y, pair_z)).wait_recv()
            oq_ref[h] = psum_ref[h] + zbuf_ref[h, 1]
            if h == 0:
                pl.semaphore_wait(xy_sem, 3)
            for j, dev in enumerate(xy_devs):
                rcopy(oq_ref.at[h], xybuf_ref.at[h, j], send_sems.at[h, 3 + j],
                      xyrecv_sems.at[h, j], dev).start()
            st = pltpu.make_async_copy(
                oq_ref.at[h],
                out_hbm.at[pl.ds(c * QM + h * HM, HM), :],
                ostore_sems.at[h, 0])
            st.start()
            ostores.append(st)

        src_cols = (
            2 * (1 - my_x) + (1 - my_y),
            2 * (1 - my_x) + my_y,
            2 * my_x + (1 - my_y),
        )
        for h in range(WAVES):
            for j, (dev, cc) in enumerate(zip(xy_devs, src_cols)):
                rcopy(oq_ref.at[h], xybuf_ref.at[h, j], send_sems.at[h, 3 + j],
                      xyrecv_sems.at[h, j], dev).wait_recv()
                st = pltpu.make_async_copy(
                    xybuf_ref.at[h, j],
                    out_hbm.at[pl.ds(cc * QM + h * HM, HM), :],
                    ostore_sems.at[h, 1 + j])
                st.start()
                ostores.append(st)

        for st in ostores:
            st.wait()
        for h in range(WAVES):
            for j, dev in enumerate(xy_devs):
                rcopy(oq_ref.at[h], xybuf_ref.at[h, j], send_sems.at[h, 3 + j],
                      xyrecv_sems.at[h, j], dev).wait_send()

            @pl.when(is_mid)
            def _(h=h):
                rcopy(psum_ref.at[h], zbuf_ref.at[h, 1], send_sems.at[h, 1],
                      zrecv_sems.at[h, 1], (my_x, my_y, other_mid)).wait_send()
                rcopy(psum_ref.at[h], zbuf_ref.at[h, 1], send_sems.at[h, 2],
                      zrecv_sems.at[h, 1], (my_x, my_y, far_edge)).wait_send()

    return pl.pallas_call(
        body,
        out_shape=jax.ShapeDtypeStruct((m, n), jnp.float32),
        in_specs=[
            pl.BlockSpec(memory_space=pl.ANY),
            pl.BlockSpec(memory_space=pl.ANY),
        ],
        out_specs=pl.BlockSpec(memory_space=pl.ANY),
        scratch_shapes=[
            pltpu.VMEM((QM, k), jnp.float32),
            pltpu.VMEM((n, k), jnp.float32),
            pltpu.VMEM((WAVES, HM, n), jnp.float32),
            pltpu.VMEM((WAVES, HM, n), jnp.float32),
            pltpu.VMEM((WAVES, HM, n), jnp.float32),
            pltpu.VMEM((WAVES, 2, HM, n), jnp.float32),
            pltpu.VMEM((WAVES, 3, HM, n), jnp.float32),
            pltpu.SemaphoreType.DMA((2,)),
            pltpu.SemaphoreType.DMA((WAVES, 6)),
            pltpu.SemaphoreType.DMA((WAVES, 2)),
            pltpu.SemaphoreType.DMA((WAVES, 3)),
            pltpu.SemaphoreType.DMA((WAVES, 4)),
            pltpu.SemaphoreType.REGULAR,
            pltpu.SemaphoreType.REGULAR,
        ],
        compiler_params=pltpu.CompilerParams(collective_id=0),
    )(dy, W)


# device time: 12030 ns/iter; 3.8423x vs baseline; 1.9470x over previous
import os

import jax
import jax.numpy as jnp
from jax import lax
from jax.experimental import pallas as pl
from jax.experimental.pallas import tpu as pltpu

ABLATE = int(os.environ.get("ABLATE", "0"))
N_Z = 4
QM = 128
WAVES = 4
HM = QM // WAVES


def kernel(dy, W):
    m, k = dy.shape
    n = W.shape[0]

    def body(dy_hbm, w_hbm, out_hbm, dyq_ref, w_ref, acc_ref, psum_ref,
             oq_ref, zbuf_ref, xybuf_ref, in_sems, send_sems, zrecv_sems,
             xyrecv_sems, ostore_sems, zr2_sem, xy_sem):
        my_x = lax.axis_index("x")
        my_y = lax.axis_index("y")
        my_z = lax.axis_index("z")
        c = 2 * my_x + my_y

        is_z0 = my_z == 0
        is_z1 = my_z == 1
        is_z3 = my_z == N_Z - 1
        is_edge = jnp.logical_or(is_z0, is_z3)
        is_mid = jnp.logical_not(is_edge)

        pair_z = my_z + jnp.where(jnp.logical_or(is_z0, my_z == 2), 1, -1)
        other_mid = jnp.where(is_z1, 2, 1)
        far_edge = jnp.where(is_z1, 3, 0)
        writer_mid = jnp.where(my_z <= 1, 2, 1)

        def rcopy(src, dst, ssem, rsem, dev):
            return pltpu.make_async_remote_copy(
                src_ref=src, dst_ref=dst, send_sem=ssem, recv_sem=rsem,
                device_id=dev, device_id_type=pl.DeviceIdType.MESH,
            )

        dcp = pltpu.make_async_copy(
            dy_hbm.at[pl.ds(c * QM, QM), :], dyq_ref, in_sems.at[0])
        wcp = pltpu.make_async_copy(w_hbm, w_ref, in_sems.at[1])
        dcp.start()
        wcp.start()

        barrier_sem = pltpu.get_barrier_semaphore()
        pl.semaphore_signal(
            barrier_sem, inc=1, device_id=(my_x, my_y, pair_z),
            device_id_type=pl.DeviceIdType.MESH,
        )
        if ABLATE != 1:
            pl.semaphore_signal(
                zr2_sem, inc=1, device_id=(my_x, my_y, writer_mid),
                device_id_type=pl.DeviceIdType.MESH,
            )
        xy_devs = (
            (1 - my_x, 1 - my_y, my_z),
            (1 - my_x, my_y, my_z),
            (my_x, 1 - my_y, my_z),
        )
        for dev in xy_devs:
            pl.semaphore_signal(
                xy_sem, inc=1, device_id=dev,
                device_id_type=pl.DeviceIdType.MESH,
            )

        dcp.wait()
        wcp.wait()

        for h in range(WAVES):
            acc_ref[h] = lax.dot_general(
                dyq_ref[pl.ds(h * HM, HM), :],
                w_ref[...],
                dimension_numbers=(((1,), (1,)), ((), ())),
                preferred_element_type=jnp.float32,
            )
            if h == 0:
                pl.semaphore_wait(barrier_sem, 1)
            if ABLATE != 1:
                rcopy(acc_ref.at[h], zbuf_ref.at[h, 0], send_sems.at[h, 0],
                      zrecv_sems.at[h, 0], (my_x, my_y, pair_z)).start()

        for h in range(WAVES) if ABLATE != 1 else ():
            rcopy(acc_ref.at[h], zbuf_ref.at[h, 0], send_sems.at[h, 0],
                  zrecv_sems.at[h, 0], (my_x, my_y, pair_z)).wait()
            psum_ref[h] = acc_ref[h] + zbuf_ref[h, 0]

            @pl.when(is_mid)
            def _(h=h):
                if h == 0:
                    pl.semaphore_wait(zr2_sem, 2)
                rcopy(psum_ref.at[h], zbuf_ref.at[h, 1], send_sems.at[h, 2],
                      zrecv_sems.at[h, 1], (my_x, my_y, far_edge)).start()
                rcopy(psum_ref.at[h], zbuf_ref.at[h, 1], send_sems.at[h, 1],
                      zrecv_sems.at[h, 1], (my_x, my_y, other_mid)).start()

        ostores = []
        for h in range(WAVES):
            if ABLATE == 1:
                oq_ref[h] = acc_ref[h]
            else:
                rcopy(psum_ref.at[h], zbuf_ref.at[h, 1], send_sems.at[h, 1],
                      zrecv_sems.at[h, 1], (my_x, my_y, pair_z)).wait_recv()
                oq_ref[h] = psum_ref[h] + zbuf_ref[h, 1]
            if h == 0:
                pl.semaphore_wait(xy_sem, 3)
            if ABLATE == 0:
                for j, dev in enumerate(xy_devs):
                    rcopy(oq_ref.at[h], xybuf_ref.at[h, j],
                          send_sems.at[h, 3 + j],
                          xyrecv_sems.at[h, j], dev).start()
            st = pltpu.make_async_copy(
                oq_ref.at[h],
                out_hbm.at[pl.ds(c * QM + h * HM, HM), :],
                ostore_sems.at[h, 0])
            st.start()
            ostores.append(st)

        src_cols = (
            2 * (1 - my_x) + (1 - my_y),
            2 * (1 - my_x) + my_y,
            2 * my_x + (1 - my_y),
        )
        for h in range(WAVES):
            for j, (dev, cc) in enumerate(zip(xy_devs, src_cols)):
                if ABLATE == 0:
                    rcopy(oq_ref.at[h], xybuf_ref.at[h, j],
                          send_sems.at[h, 3 + j],
                          xyrecv_sems.at[h, j], dev).wait_recv()
                src = xybuf_ref.at[h, j] if ABLATE == 0 else oq_ref.at[h]
                st = pltpu.make_async_copy(
                    src,
                    out_hbm.at[pl.ds(cc * QM + h * HM, HM), :],
                    ostore_sems.at[h, 1 + j])
                st.start()
                ostores.append(st)

        for st in ostores:
            st.wait()
        for h in range(WAVES) if ABLATE == 0 else ():
            for j, dev in enumerate(xy_devs):
                rcopy(oq_ref.at[h], xybuf_ref.at[h, j], send_sems.at[h, 3 + j],
                      xyrecv_sems.at[h, j], dev).wait_send()

            @pl.when(is_mid)
            def _(h=h):
                rcopy(psum_ref.at[h], zbuf_ref.at[h, 1], send_sems.at[h, 1],
                      zrecv_sems.at[h, 1], (my_x, my_y, other_mid)).wait_send()
                rcopy(psum_ref.at[h], zbuf_ref.at[h, 1], send_sems.at[h, 2],
                      zrecv_sems.at[h, 1], (my_x, my_y, far_edge)).wait_send()

    return pl.pallas_call(
        body,
        out_shape=jax.ShapeDtypeStruct((m, n), jnp.float32),
        in_specs=[
            pl.BlockSpec(memory_space=pl.ANY),
            pl.BlockSpec(memory_space=pl.ANY),
        ],
        out_specs=pl.BlockSpec(memory_space=pl.ANY),
        scratch_shapes=[
            pltpu.VMEM((QM, k), jnp.float32),
            pltpu.VMEM((n, k), jnp.float32),
            pltpu.VMEM((WAVES, HM, n), jnp.float32),
            pltpu.VMEM((WAVES, HM, n), jnp.float32),
            pltpu.VMEM((WAVES, HM, n), jnp.float32),
            pltpu.VMEM((WAVES, 2, HM, n), jnp.float32),
            pltpu.VMEM((WAVES, 3, HM, n), jnp.float32),
            pltpu.SemaphoreType.DMA((2,)),
            pltpu.SemaphoreType.DMA((WAVES, 6)),
            pltpu.SemaphoreType.DMA((WAVES, 2)),
            pltpu.SemaphoreType.DMA((WAVES, 3)),
            pltpu.SemaphoreType.DMA((WAVES, 4)),
            pltpu.SemaphoreType.REGULAR,
            pltpu.SemaphoreType.REGULAR,
        ],
        compiler_params=pltpu.CompilerParams(collective_id=0),
    )(dy, W)
